# baseline (speedup 1.0000x reference)
import functools

import jax
import jax.numpy as jnp
from jax import lax
from jax.experimental import pallas as pl
from jax.experimental.pallas import tpu as pltpu
from jax.experimental.pallas import tpu_sc as plsc

BATCH = 16384
FEATS = 2048
LANES = 16
NGRP = FEATS // LANES
NUM_WORKERS = 32
RBLK = 8
NBLK = BATCH // NUM_WORKERS // RBLK
BLKW = RBLK * FEATS
NBUF = 2


def _permute_sc(inputs, perm):
  mesh = plsc.VectorSubcoreMesh(core_axis_name="c", subcore_axis_name="s")

  @functools.partial(
      pl.kernel,
      mesh=mesh,
      out_type=jax.ShapeDtypeStruct((BATCH, FEATS), jnp.float32),
      scratch_types=[
          pltpu.VMEM((FEATS,), jnp.int32),
          [pltpu.VMEM((BLKW,), jnp.float32) for _ in range(NBUF)],
          [pltpu.VMEM((BLKW,), jnp.float32) for _ in range(NBUF)],
          [pltpu.SemaphoreType.DMA for _ in range(NBUF)],
          [pltpu.SemaphoreType.DMA for _ in range(NBUF)],
      ],
      compiler_params=pltpu.CompilerParams(
          use_tc_tiling_on_sc=True, needs_layout_passes=False),
  )
  def k(in_hbm, perm_hbm, out_hbm, idx_v, in_v, out_v, in_sem, out_sem):
    wid = lax.axis_index("s") * 2 + lax.axis_index("c")
    row_base = wid * NBLK * RBLK

    pltpu.sync_copy(perm_hbm, idx_v)

    def in_copy(g, b, r):
      return pltpu.make_async_copy(
          in_hbm.at[row_base + g * RBLK + r],
          in_v[b].at[pl.ds(r * FEATS, FEATS)], in_sem[b])

    def out_copy(g, b, r):
      return pltpu.make_async_copy(
          out_v[b].at[pl.ds(r * FEATS, FEATS)],
          out_hbm.at[row_base + g * RBLK + r], out_sem[b])

    def process(g, b):
      for r in range(RBLK):
        in_copy(g, b, r).wait()

      @pl.when(g >= NBUF)
      def _():
        for r in range(RBLK):
          out_copy(g - NBUF, b, r).wait()

      for r in range(RBLK):
        rb = r * FEATS

        @plsc.parallel_loop(0, NGRP)
        def _(j):
          c0 = j * LANES
          val = plsc.load_gather(in_v[b], [idx_v[pl.ds(c0, LANES)] + rb])
          out_v[b][pl.ds(c0 + rb, LANES)] = val

        out_copy(g, b, r).start()

        @pl.when(g + NBUF < NBLK)
        def _():
          in_copy(g + NBUF, b, r).start()

    for b in range(NBUF):
      for r in range(RBLK):
        in_copy(b, b, r).start()

    def step(s, carry):
      for b in range(NBUF):
        process(s * NBUF + b, b)
      return carry

    lax.fori_loop(0, NBLK // NBUF, step, 0)
    for g in range(NBLK - NBUF, NBLK):
      for r in range(RBLK):
        out_copy(jnp.int32(g), g % NBUF, r).wait()

  return k(inputs, perm)


def kernel(inputs, permutation):
  out = _permute_sc(inputs, permutation.astype(jnp.int32))
  return (out, 0)


# speedup vs baseline: 2.3579x; 2.3579x over previous
import functools

import jax
import jax.numpy as jnp
from jax import lax
from jax.experimental import pallas as pl
from jax.experimental.pallas import tpu as pltpu
from jax.experimental.pallas import tpu_sc as plsc

BATCH = 16384
FEATS = 2048
LANES = 16
NUM_WORKERS = 32
RBLK = 8
NBLK = BATCH // NUM_WORKERS // RBLK
BLKW = RBLK * FEATS
CHUNK = 256
NCH = CHUNK // LANES
NBUF = 2
NFULL = (NBLK // NBUF) * NBUF


def _permute_sc(inputs, perm):
  mesh = plsc.VectorSubcoreMesh(core_axis_name="c", subcore_axis_name="s")

  @functools.partial(
      pl.kernel,
      mesh=mesh,
      out_type=jax.ShapeDtypeStruct((BATCH, FEATS), jnp.float32),
      scratch_types=[
          pltpu.VMEM((FEATS,), jnp.int32),
          [pltpu.VMEM((BLKW,), jnp.float32) for _ in range(NBUF)],
          [pltpu.VMEM((BLKW,), jnp.float32) for _ in range(NBUF)],
          [pltpu.SemaphoreType.DMA for _ in range(NBUF)],
          [pltpu.SemaphoreType.DMA for _ in range(NBUF)],
      ],
      compiler_params=pltpu.CompilerParams(
          use_tc_tiling_on_sc=True, needs_layout_passes=False),
  )
  def k(in_hbm, perm_hbm, out_hbm, idx_v, in_v, out_v, in_sem, out_sem):
    wid = lax.axis_index("s") * 2 + lax.axis_index("c")
    row_base = wid * NBLK * RBLK

    pltpu.sync_copy(perm_hbm, idx_v)

    def in_copies(g, b):
      row0 = row_base + g * RBLK
      return [
          pltpu.make_async_copy(
              in_hbm.at[row0 + r], in_v[b].at[pl.ds(r * FEATS, FEATS)],
              in_sem[b]) for r in range(RBLK)
      ]

    def out_copies(g, b):
      row0 = row_base + g * RBLK
      return [
          pltpu.make_async_copy(
              out_v[b].at[pl.ds(r * FEATS, FEATS)], out_hbm.at[row0 + r],
              out_sem[b]) for r in range(RBLK)
      ]

    def process(g, b):
      for c in in_copies(g, b):
        c.wait()

      @pl.when(g >= NBUF)
      def _():
        for c in out_copies(g - NBUF, b):
          c.wait()

      for m in range(FEATS // CHUNK):
        pv = tuple(
            idx_v[pl.ds(m * CHUNK + j * LANES, LANES)] for j in range(NCH))

        @plsc.parallel_loop(0, RBLK, carry=pv)
        def _(r, pvecs):
          rb = r * FEATS
          vals = [
              plsc.load_gather(in_v[b], [pvecs[j] + rb]) for j in range(NCH)
          ]
          for j in range(NCH):
            out_v[b][pl.ds(m * CHUNK + j * LANES + rb, LANES)] = vals[j]
          return pvecs

      for c in out_copies(g, b):
        c.start()

      @pl.when(g + NBUF < NBLK)
      def _():
        for c in in_copies(g + NBUF, b):
          c.start()

    for b in range(NBUF):
      for c in in_copies(b, b):
        c.start()

    def step(s, carry):
      for b in range(NBUF):
        process(s * NBUF + b, b)
      return carry

    lax.fori_loop(0, NBLK // NBUF, step, 0)
    for g in range(NFULL, NBLK):
      process(jnp.int32(g), g % NBUF)
    for g in range(NBLK - NBUF, NBLK):
      for c in out_copies(jnp.int32(g), g % NBUF):
        c.wait()

  return k(inputs, perm)


def kernel(inputs, permutation):
  out = _permute_sc(inputs, permutation.astype(jnp.int32))
  return (out, 0)
